# baseline (speedup 1.0000x reference)
import functools

import jax
import jax.numpy as jnp
from jax import lax
from jax.experimental import pallas as pl
from jax.experimental.pallas import tpu as pltpu
from jax.experimental.pallas import tpu_sc as plsc

NUSER = 1000000
NITEM = 100000
HID = 64
BATCH = 4096

_info = plsc.get_sparse_core_info()
_NC, _NS, _NL = _info.num_cores, _info.num_subcores, _info.num_lanes
_NW = _NC * _NS
_BPW = BATCH // _NW
_NBUF = 4


@functools.partial(
    pl.kernel,
    mesh=plsc.VectorSubcoreMesh(core_axis_name="c", subcore_axis_name="s"),
    out_type=jax.ShapeDtypeStruct((HID, BATCH), jnp.float32),
    scratch_types=(
        [pltpu.VMEM((_BPW,), jnp.int32)]
        + [pltpu.VMEM((HID, 128), jnp.float32)] * _NBUF
        + [pltpu.VMEM((HID, _BPW), jnp.float32)]
        + [pltpu.SemaphoreType.DMA]
    ),
    compiler_params=pltpu.CompilerParams(
        needs_layout_passes=False, disable_bounds_checks=True
    ),
)
def _user_lookup(user_hbm, uft_hbm, uout_hbm,
                 uidx_v, blk0, blk1, blk2, blk3, cols_v, sem):
    blks = (blk0, blk1, blk2, blk3)
    wid = lax.axis_index("s") * _NC + lax.axis_index("c")
    base = pl.multiple_of(wid * _BPW, _BPW)
    pltpu.sync_copy(user_hbm.at[pl.ds(base, _BPW)], uidx_v)
    lanes = lax.iota(jnp.int32, _NL)

    def extract(blk, rr, r):
        rr_v = jnp.full((_NL,), rr, jnp.int32)
        r_v = jnp.full((_NL,), r, jnp.int32)
        for k in range(HID // _NL):
            cvec = k * _NL + lanes
            val = plsc.load_gather(blk, [cvec, rr_v])
            plsc.store_scatter(cols_v, [cvec, r_v], val)

    def group(g):
        vec = uidx_v[pl.ds(g * 16, 16)]
        for h in range(16 // _NBUF):
            handles = []
            for b in range(_NBUF):
                r0 = pl.multiple_of((vec[h * _NBUF + b] >> 7) * 128, 128)
                handles.append(pltpu.async_copy(
                    uft_hbm.at[:, pl.ds(r0, 128)], blks[b], sem))
            for b in range(_NBUF):
                j = h * _NBUF + b
                handles[b].wait()
                extract(blks[b], vec[j] & 127, g * 16 + j)

    pl.loop(0, _BPW // 16)(group)
    pltpu.async_copy(cols_v, uout_hbm.at[:, pl.ds(base, _BPW)], sem).wait()


def _item_tc_body(idx_ref, tbl_ref, out_ref):
    g = pl.program_id(0)
    j = pl.program_id(1)
    c = idx_ref[g * 128 + j] & 127
    lanes = lax.broadcasted_iota(jnp.int32, (HID, 128), 1)
    col = jnp.sum(jnp.where(lanes == c, tbl_ref[...], 0.0), axis=1,
                  keepdims=True)
    out_ref[...] = jnp.where(lanes == j, col, out_ref[...])


def _item_gather_tc(item_idx, ift):
    return pl.pallas_call(
        _item_tc_body,
        grid_spec=pltpu.PrefetchScalarGridSpec(
            num_scalar_prefetch=1,
            grid=(BATCH // 128, 128),
            in_specs=[pl.BlockSpec(
                (HID, 128), lambda g, j, idx: (0, idx[g * 128 + j] >> 7))],
            out_specs=pl.BlockSpec((HID, 128), lambda g, j, idx: (0, g)),
        ),
        out_shape=jax.ShapeDtypeStruct((HID, BATCH), jnp.float32),
    )(item_idx, ift)


def kernel(user, item, user_factor, item_factor):
    user = user.astype(jnp.int32)
    item = item.astype(jnp.int32)
    uout_t = _user_lookup(user, user_factor.T)
    iout_t = _item_gather_tc(item, item_factor.T)
    return (uout_t.T, iout_t.T)


# speedup vs baseline: 14.2163x; 14.2163x over previous
import functools

import jax
import jax.numpy as jnp
from jax import lax
from jax.experimental import pallas as pl
from jax.experimental.pallas import tpu as pltpu
from jax.experimental.pallas import tpu_sc as plsc

NUSER = 1000000
NITEM = 100000
HID = 64
BATCH = 4096

_info = plsc.get_sparse_core_info()
_NC, _NS, _NL = _info.num_cores, _info.num_subcores, _info.num_lanes
_NW = _NC * _NS
_BPW = BATCH // _NW
_NBUF = 8


@functools.partial(
    pl.kernel,
    mesh=plsc.VectorSubcoreMesh(core_axis_name="c", subcore_axis_name="s"),
    out_type=jax.ShapeDtypeStruct((HID, BATCH), jnp.float32),
    scratch_types=(
        [pltpu.VMEM((_BPW,), jnp.int32)]
        + [pltpu.VMEM((HID, 128), jnp.float32)] * _NBUF
        + [pltpu.VMEM((HID, _BPW), jnp.float32)]
        + [pltpu.SemaphoreType.DMA]
    ),
    compiler_params=pltpu.CompilerParams(
        needs_layout_passes=False, disable_bounds_checks=True
    ),
)
def _user_lookup(user_hbm, uft_hbm, uout_hbm,
                 uidx_v, blk0, blk1, blk2, blk3, blk4, blk5, blk6, blk7,
                 cols_v, sem):
    blks = (blk0, blk1, blk2, blk3, blk4, blk5, blk6, blk7)
    wid = lax.axis_index("s") * _NC + lax.axis_index("c")
    base = pl.multiple_of(wid * _BPW, _BPW)
    pltpu.sync_copy(user_hbm.at[pl.ds(base, _BPW)], uidx_v)
    lanes = lax.iota(jnp.int32, _NL)

    def extract(blk, rr, r):
        rr_v = jnp.full((_NL,), rr, jnp.int32)
        r_v = jnp.full((_NL,), r, jnp.int32)
        for k in range(HID // _NL):
            cvec = k * _NL + lanes
            val = plsc.load_gather(blk, [cvec, rr_v])
            plsc.store_scatter(cols_v, [cvec, r_v], val)

    def fire(vec, h, b):
        r0 = pl.multiple_of((vec[h] >> 7) * 128, 128)
        pltpu.async_copy(uft_hbm.at[:, pl.ds(r0, 128)], blks[b], sem)

    vec0 = uidx_v[pl.ds(0, 16)]
    for b in range(_NBUF):
        fire(vec0, b, b)

    def group(g):
        vec = uidx_v[pl.ds(g * 16, 16)]
        nxt = uidx_v[pl.ds(jnp.minimum(g + 1, _BPW // 16 - 1) * 16, 16)]
        last = g == _BPW // 16 - 1
        for h in range(16):
            b = h % _NBUF
            pltpu.make_async_copy(
                uft_hbm.at[:, pl.ds(0, 128)], blks[b], sem).wait()
            extract(blks[b], vec[h] & 127, g * 16 + h)
            if h < 16 - _NBUF:
                fire(vec, h + _NBUF, b)
            else:
                @pl.when(jnp.logical_not(last))
                def _():
                    fire(nxt, h + _NBUF - 16, b)

    pl.loop(0, _BPW // 16)(group)
    pltpu.async_copy(cols_v, uout_hbm.at[:, pl.ds(base, _BPW)], sem).wait()


@functools.partial(
    pl.kernel,
    mesh=plsc.VectorSubcoreMesh(core_axis_name="c", subcore_axis_name="s"),
    out_type=jax.ShapeDtypeStruct((BATCH, HID), jnp.float32),
    scratch_types=[
        pltpu.VMEM((_BPW,), jnp.int32),
        pltpu.VMEM((_BPW, HID), jnp.float32),
        pltpu.SemaphoreType.DMA,
    ],
    compiler_params=pltpu.CompilerParams(use_tc_tiling_on_sc=False),
)
def _item_lookup(item_hbm, if_hbm, iout_hbm, iidx_v, rows_v, sem):
    wid = lax.axis_index("s") * _NC + lax.axis_index("c")
    base = wid * _BPW
    pltpu.sync_copy(item_hbm.at[pl.ds(base, _BPW)], iidx_v)
    pltpu.async_copy(if_hbm.at[iidx_v], rows_v, sem).wait()
    pltpu.async_copy(rows_v, iout_hbm.at[pl.ds(base, _BPW)], sem).wait()


def kernel(user, item, user_factor, item_factor):
    user = user.astype(jnp.int32)
    item = item.astype(jnp.int32)
    uout_t = _user_lookup(user, user_factor.T)
    item_emb = _item_lookup(item, item_factor)
    return (uout_t.T, item_emb)


# speedup vs baseline: 16.4265x; 1.1555x over previous
import functools

import jax
import jax.numpy as jnp
from jax import lax
from jax.experimental import pallas as pl
from jax.experimental.pallas import tpu as pltpu
from jax.experimental.pallas import tpu_sc as plsc

NUSER = 1000000
NITEM = 100000
HID = 64
BATCH = 4096

_info = plsc.get_sparse_core_info()
_NC, _NS, _NL = _info.num_cores, _info.num_subcores, _info.num_lanes
_NW = _NC * _NS
_BPW = BATCH // _NW
_NBUF = 4
_NG = _BPW // 16


@functools.partial(
    pl.kernel,
    mesh=plsc.VectorSubcoreMesh(core_axis_name="c", subcore_axis_name="s"),
    out_type=[
        jax.ShapeDtypeStruct((HID, BATCH), jnp.float32),
        jax.ShapeDtypeStruct((HID, BATCH), jnp.float32),
    ],
    scratch_types=(
        [pltpu.VMEM((_BPW,), jnp.int32)] * 2
        + [pltpu.VMEM((HID, 128), jnp.float32)] * (2 * _NBUF)
        + [pltpu.VMEM((HID, _BPW), jnp.float32)] * 2
        + [pltpu.SemaphoreType.DMA] * 2
    ),
    compiler_params=pltpu.CompilerParams(
        needs_layout_passes=False, disable_bounds_checks=True
    ),
)
def _lookup(user_hbm, item_hbm, uft_hbm, ift_hbm, uout_hbm, iout_hbm,
            uidx_v, iidx_v,
            ublk0, ublk1, ublk2, ublk3, iblk0, iblk1, iblk2, iblk3,
            ucols_v, icols_v, usem, isem):
    ublks = (ublk0, ublk1, ublk2, ublk3)
    iblks = (iblk0, iblk1, iblk2, iblk3)
    wid = lax.axis_index("s") * _NC + lax.axis_index("c")
    base = pl.multiple_of(wid * _BPW, _BPW)
    pltpu.sync_copy(user_hbm.at[pl.ds(base, _BPW)], uidx_v)
    pltpu.sync_copy(item_hbm.at[pl.ds(base, _BPW)], iidx_v)
    lanes = lax.iota(jnp.int32, _NL)

    def extract(blk, cols, rr, r):
        rr_v = jnp.full((_NL,), rr, jnp.int32)
        r_v = jnp.full((_NL,), r, jnp.int32)
        for k in range(HID // _NL):
            cvec = k * _NL + lanes
            val = plsc.load_gather(blk, [cvec, rr_v])
            plsc.store_scatter(cols, [cvec, r_v], val)

    def fire(tbl, blk, sem, vec, h):
        r0 = pl.multiple_of((vec[h] >> 7) * 128, 128)
        pltpu.async_copy(tbl.at[:, pl.ds(r0, 128)], blk, sem)

    uvec0 = uidx_v[pl.ds(0, 16)]
    ivec0 = iidx_v[pl.ds(0, 16)]
    for b in range(_NBUF):
        fire(uft_hbm, ublks[b], usem, uvec0, b)
        fire(ift_hbm, iblks[b], isem, ivec0, b)

    def group(g):
        uvec = uidx_v[pl.ds(g * 16, 16)]
        ivec = iidx_v[pl.ds(g * 16, 16)]
        gn = jnp.minimum(g + 1, _NG - 1) * 16
        unxt = uidx_v[pl.ds(gn, 16)]
        inxt = iidx_v[pl.ds(gn, 16)]
        not_last = g < _NG - 1
        for h in range(16):
            b = h % _NBUF
            r = g * 16 + h
            pltpu.make_async_copy(
                uft_hbm.at[:, pl.ds(0, 128)], ublks[b], usem).wait()
            extract(ublks[b], ucols_v, uvec[h] & 127, r)
            if h < 16 - _NBUF:
                fire(uft_hbm, ublks[b], usem, uvec, h + _NBUF)
            else:
                @pl.when(not_last)
                def _():
                    fire(uft_hbm, ublks[b], usem, unxt, h + _NBUF - 16)
            pltpu.make_async_copy(
                ift_hbm.at[:, pl.ds(0, 128)], iblks[b], isem).wait()
            extract(iblks[b], icols_v, ivec[h] & 127, r)
            if h < 16 - _NBUF:
                fire(ift_hbm, iblks[b], isem, ivec, h + _NBUF)
            else:
                @pl.when(not_last)
                def _():
                    fire(ift_hbm, iblks[b], isem, inxt, h + _NBUF - 16)

    pl.loop(0, _NG)(group)
    uw = pltpu.async_copy(ucols_v, uout_hbm.at[:, pl.ds(base, _BPW)], usem)
    iw = pltpu.async_copy(icols_v, iout_hbm.at[:, pl.ds(base, _BPW)], isem)
    uw.wait()
    iw.wait()


def kernel(user, item, user_factor, item_factor):
    user = user.astype(jnp.int32)
    item = item.astype(jnp.int32)
    uout_t, iout_t = _lookup(user, item, user_factor.T, item_factor.T)
    return (uout_t.T, iout_t.T)
